# baseline (speedup 1.0000x reference)
import jax
import jax.numpy as jnp
from jax import lax
from jax.experimental import pallas as pl
from jax.experimental.pallas import tpu as pltpu
from jax.experimental.pallas import tpu_sc as plsc

NC = 2
NS = 16
NW = NC * NS

BATCH, HIST, D = 4096, 200, 64
B_PER_W = BATCH // NW
L = 16


def _gather_kernel(table_hbm, idx_hbm, out_hbm, idx_v, idx_t, rows_v, tbuf, gsem, osem):
    wid = lax.axis_index("s") * NC + lax.axis_index("c")
    b0 = wid * B_PER_W
    pltpu.sync_copy(idx_hbm.at[pl.ds(b0, B_PER_W)], idx_v)

    lanes = lax.iota(jnp.int32, L)

    @pl.loop(0, HIST)
    def tr_idx(t):
        tvec = jnp.full((L,), t, jnp.int32)

        @pl.loop(0, B_PER_W // L)
        def tr_idx_c(c):
            v = plsc.load_gather(idx_v, [c * L + lanes, tvec])
            idx_t[t, pl.ds(c * L, L)] = v

    def issue_rows(t, buf):
        @pl.loop(0, B_PER_W // L)
        def chunk(c):
            v = idx_t[t, pl.ds(c * L, L)]
            for l in range(L):
                pltpu.async_copy(
                    table_hbm.at[0, pl.ds(v[l], 1)],
                    rows_v.at[buf, pl.ds(c * L + l, 1)],
                    gsem.at[buf],
                )

    def drain_rows(buf):
        pltpu.make_async_copy(
            table_hbm.at[0, pl.ds(0, B_PER_W)], rows_v.at[buf], gsem.at[buf]
        ).wait()

    def transpose_rows(buf):
        @pl.loop(0, D)
        def tr_d(d):
            dvec = jnp.full((L,), d, jnp.int32)

            @pl.loop(0, B_PER_W // L)
            def tr_c(c):
                v = plsc.load_gather(rows_v.at[buf], [c * L + lanes, dvec])
                tbuf[buf, d, pl.ds(c * L, L)] = v

    def start_out(t, buf):
        pltpu.async_copy(
            tbuf.at[buf], out_hbm.at[t, :, pl.ds(b0, B_PER_W)], osem.at[buf]
        )

    def wait_out(t, buf):
        pltpu.make_async_copy(
            tbuf.at[buf], out_hbm.at[t, :, pl.ds(b0, B_PER_W)], osem.at[buf]
        ).wait()

    issue_rows(0, 0)
    issue_rows(1, 1)

    @pl.loop(0, HIST - 2, step=2)
    def step(t):
        drain_rows(0)

        @pl.when(t > 0)
        def _():
            wait_out(t - 2, 0)

        transpose_rows(0)
        start_out(t, 0)
        issue_rows(t + 2, 0)
        drain_rows(1)

        @pl.when(t > 0)
        def _():
            wait_out(t - 1, 1)

        transpose_rows(1)
        start_out(t + 1, 1)
        issue_rows(t + 3, 1)

    t = HIST - 2
    drain_rows(0)
    wait_out(t - 2, 0)
    transpose_rows(0)
    start_out(t, 0)
    drain_rows(1)
    wait_out(t - 1, 1)
    transpose_rows(1)
    start_out(t + 1, 1)
    wait_out(t, 0)
    wait_out(t + 1, 1)


def kernel(words, table):
    idx = words.astype(jnp.int32)
    mesh = plsc.VectorSubcoreMesh(core_axis_name="c", subcore_axis_name="s")
    f = pl.kernel(
        _gather_kernel,
        out_type=jax.ShapeDtypeStruct((HIST, D, BATCH), jnp.float32),
        mesh=mesh,
        scratch_types=[
            pltpu.VMEM((B_PER_W, HIST), jnp.int32),
            pltpu.VMEM((HIST, B_PER_W), jnp.int32),
            pltpu.VMEM((2, B_PER_W, D), jnp.float32),
            pltpu.VMEM((2, D, B_PER_W), jnp.float32),
            pltpu.SemaphoreType.DMA((2,)),
            pltpu.SemaphoreType.DMA((2,)),
        ],
        compiler_params=pltpu.CompilerParams(
            use_tc_tiling_on_sc=True, needs_layout_passes=False
        ),
    )
    tb = table.reshape(1, 1000000, D)
    out = f(tb, idx)
    return lax.optimization_barrier(out.transpose(2, 0, 1))


# speedup vs baseline: 2.4001x; 2.4001x over previous
import jax
import jax.numpy as jnp
from jax import lax
from jax.experimental import pallas as pl
from jax.experimental.pallas import tpu as pltpu
from jax.experimental.pallas import tpu_sc as plsc

NC = 2
NS = 16
NW = NC * NS

BATCH, HIST, D = 4096, 200, 64
CHUNKS = 1
B_CHUNK = BATCH // CHUNKS
B_PER_W = B_CHUNK // NW


def _gather_kernel(table_hbm, idx_hbm, out_hbm, idx_v, rows_v, gsem, osem):
    wid = lax.axis_index("s") * NC + lax.axis_index("c")
    b0 = wid * B_PER_W
    pltpu.sync_copy(
        idx_hbm.at[pl.ds(b0 * HIST, B_PER_W * HIST)],
        idx_v.at[pl.ds(0, B_PER_W * HIST)],
    )

    def issue_rows(b, buf):
        base = b * HIST

        @pl.loop(0, HIST // 8)
        def chunk(c):
            v = idx_v[pl.ds(base + c * 8, 16)]
            for l in range(8):
                pltpu.async_copy(
                    table_hbm.at[0, pl.ds(v[l], 1)],
                    rows_v.at[buf, pl.ds(c * 8 + l, 1)],
                    gsem.at[buf],
                )

    def drain_rows(buf):
        pltpu.make_async_copy(
            table_hbm.at[0, pl.ds(0, HIST)], rows_v.at[buf], gsem.at[buf]
        ).wait()

    def start_out(b, buf):
        pltpu.async_copy(rows_v.at[buf], out_hbm.at[b0 + b], osem.at[buf])

    def wait_out(b, buf):
        pltpu.make_async_copy(
            rows_v.at[buf], out_hbm.at[b0 + b], osem.at[buf]
        ).wait()

    issue_rows(0, 0)
    issue_rows(1, 1)

    @pl.loop(0, B_PER_W - 2, step=2)
    def step(b):
        drain_rows(0)
        start_out(b, 0)
        drain_rows(1)
        start_out(b + 1, 1)
        wait_out(b, 0)
        issue_rows(b + 2, 0)
        wait_out(b + 1, 1)
        issue_rows(b + 3, 1)

    b = B_PER_W - 2
    drain_rows(0)
    start_out(b, 0)
    drain_rows(1)
    start_out(b + 1, 1)
    wait_out(b, 0)
    wait_out(b + 1, 1)


def kernel(words, table):
    idx = words.reshape(BATCH * HIST).astype(jnp.int32)
    mesh = plsc.VectorSubcoreMesh(core_axis_name="c", subcore_axis_name="s")
    f = pl.kernel(
        _gather_kernel,
        out_type=jax.ShapeDtypeStruct((B_CHUNK, HIST, D), jnp.float32),
        mesh=mesh,
        scratch_types=[
            pltpu.VMEM((B_PER_W * HIST + 16,), jnp.int32),
            pltpu.VMEM((2, HIST, D), jnp.float32),
            pltpu.SemaphoreType.DMA((2,)),
            pltpu.SemaphoreType.DMA((2,)),
        ],
        compiler_params=pltpu.CompilerParams(use_tc_tiling_on_sc=True),
    )
    tb = table.reshape(1, 1000000, 64)
    out = f(tb, idx)
    return lax.optimization_barrier(out)


# speedup vs baseline: 2.4059x; 1.0024x over previous
import jax
import jax.numpy as jnp
from jax import lax
from jax.experimental import pallas as pl
from jax.experimental.pallas import tpu as pltpu
from jax.experimental.pallas import tpu_sc as plsc

NC = 2
NS = 16
NW = NC * NS

BATCH, HIST, D = 4096, 200, 64
B_PER_W = BATCH // NW


def _gather_kernel(table_hbm, idx_hbm, out_hbm, idx_v, rows_v, gsem, osem):
    wid = lax.axis_index("s") * NC + lax.axis_index("c")
    b0 = wid * B_PER_W
    pltpu.sync_copy(
        idx_hbm.at[pl.ds(b0 * HIST, B_PER_W * HIST)],
        idx_v.at[pl.ds(0, B_PER_W * HIST)],
    )

    def issue_rows(b, buf):
        base = b * HIST

        @pl.loop(0, HIST // 8)
        def chunk(c):
            v = idx_v[pl.ds(base + c * 8, 16)]
            for l in range(8):
                pltpu.async_copy(
                    table_hbm.at[0, pl.ds(v[l], 1)],
                    rows_v.at[buf, pl.ds(c * 8 + l, 1)],
                    gsem.at[buf],
                )

    def drain_rows(buf):
        pltpu.make_async_copy(
            table_hbm.at[0, pl.ds(0, HIST)], rows_v.at[buf], gsem.at[buf]
        ).wait()

    def start_out(b, buf):
        pltpu.async_copy(rows_v.at[buf], out_hbm.at[b0 + b], osem.at[buf])

    def wait_out(b, buf):
        pltpu.make_async_copy(
            rows_v.at[buf], out_hbm.at[b0 + b], osem.at[buf]
        ).wait()

    issue_rows(0, 0)
    issue_rows(1, 1)

    @pl.loop(0, B_PER_W - 2, step=2)
    def step(b):
        drain_rows(0)
        start_out(b, 0)
        drain_rows(1)
        start_out(b + 1, 1)
        wait_out(b, 0)
        issue_rows(b + 2, 0)
        wait_out(b + 1, 1)
        issue_rows(b + 3, 1)

    b = B_PER_W - 2
    drain_rows(0)
    start_out(b, 0)
    drain_rows(1)
    start_out(b + 1, 1)
    wait_out(b, 0)
    wait_out(b + 1, 1)


def kernel(words, table):
    idx = words.reshape(BATCH * HIST).astype(jnp.int32)
    mesh = plsc.VectorSubcoreMesh(core_axis_name="c", subcore_axis_name="s")
    f = pl.kernel(
        _gather_kernel,
        out_type=jax.ShapeDtypeStruct((BATCH, HIST, D), jnp.float32),
        mesh=mesh,
        scratch_types=[
            pltpu.VMEM((B_PER_W * HIST + 16,), jnp.int32),
            pltpu.VMEM((2, HIST, D), jnp.float32),
            pltpu.SemaphoreType.DMA((2,)),
            pltpu.SemaphoreType.DMA((2,)),
        ],
        compiler_params=pltpu.CompilerParams(use_tc_tiling_on_sc=True),
    )
    tb = table.reshape(1, 1000000, D)
    out = f(tb, idx)
    return lax.optimization_barrier(out)
